# Initial kernel scaffold; baseline (speedup 1.0000x reference)
import jax
import jax.numpy as jnp
from jax.experimental import pallas as pl


def kernel(input, mask, shift_sz, stride, triple_w, flag):
    raise NotImplementedError("write your pallas kernel here")


# speedup vs baseline: 1.1775x; 1.1775x over previous
import jax
import jax.numpy as jnp
from jax.experimental import pallas as pl
from jax.experimental.pallas import tpu as pltpu

H = 96
W = 96
N = H * W
CH = 64
BR = 256


def _attn_kernel(fT_blk, lT, l2d, colneg, out_blk):
    lt = lT[...]
    inv = jax.lax.rsqrt(jnp.sum(lt * lt, axis=0, keepdims=True))
    ltn = lt * inv
    logits = jax.lax.dot_general(
        fT_blk[...], ltn, (((0,), (0,)), ((), ())),
        preferred_element_type=jnp.float32)
    logits = logits + colneg[...]
    m = jnp.max(logits, axis=1, keepdims=True)
    p = jnp.exp(logits - m)
    s = jnp.sum(p, axis=1, keepdims=True)
    attn = p / s
    out_blk[...] = jax.lax.dot_general(
        attn, l2d[...], (((1,), (0,)), ((), ())),
        preferred_element_type=jnp.float32)


def kernel(input, mask, shift_sz, stride, triple_w, flag):
    bz, c, h, w = input.shape
    ch = c // 2
    fT = input[0, :ch].reshape(ch, N)
    lT = input[0, ch:c].reshape(ch, N)
    l2d = lT.T
    flag = flag.astype(jnp.int32)
    colneg = jnp.where(flag == 1, -1e30, 0.0).astype(jnp.float32).reshape(1, N)

    grid = (N // BR,)
    out = pl.pallas_call(
        _attn_kernel,
        grid=grid,
        in_specs=[
            pl.BlockSpec((ch, BR), lambda i: (0, i)),
            pl.BlockSpec((ch, N), lambda i: (0, 0)),
            pl.BlockSpec((N, ch), lambda i: (0, 0)),
            pl.BlockSpec((1, N), lambda i: (0, 0)),
        ],
        out_specs=pl.BlockSpec((BR, ch), lambda i: (i, 0)),
        out_shape=jax.ShapeDtypeStruct((N, ch), jnp.float32),
        compiler_params=pltpu.CompilerParams(
            dimension_semantics=("arbitrary",)),
    )(fT, lT, l2d, colneg)

    fm2d = jnp.where((flag == 1)[:, None], out, 0.0)
    former_masked = fm2d.T.reshape(1, ch, h, w)
    return jnp.concatenate([input, former_masked], axis=1)
